# baseline (speedup 1.0000x reference)
import functools

import jax
import jax.numpy as jnp
from jax import lax
from jax.experimental import pallas as pl
from jax.experimental.pallas import tpu as pltpu
from jax.experimental.pallas import tpu_sc as plsc

N = 2048
D_IN = 1024
HID = 1024
T = 16
B = 256
NBLK = 23
P = NBLK * B
BT_LEN = 32

NC, NS, L = 2, 16, 16
NW = NC * NS
CHUNK = N // NW


def _mesh():
    return plsc.VectorSubcoreMesh(core_axis_name="c", subcore_axis_name="s")


def _mesh1():
    return plsc.VectorSubcoreMesh(
        core_axis_name="c", subcore_axis_name="s", num_cores=1)


_SC_PARAMS = pltpu.CompilerParams(needs_layout_passes=False)


def _backbone_body(x_ref, wb_ref, bb_ref, out_ref):
    xb = x_ref[...].astype(jnp.bfloat16)
    wb = wb_ref[...].astype(jnp.bfloat16)
    acc = jnp.dot(xb, wb, preferred_element_type=jnp.float32)
    out_ref[...] = jnp.maximum(acc + bb_ref[...], 0.0)


def _backbone(x, Wb, bb2):
    return pl.pallas_call(
        _backbone_body,
        grid=(8,),
        in_specs=[
            pl.BlockSpec((N // 8, D_IN), lambda i: (i, 0)),
            pl.BlockSpec((D_IN, HID), lambda i: (0, 0)),
            pl.BlockSpec((1, HID), lambda i: (0, 0)),
        ],
        out_specs=pl.BlockSpec((N // 8, HID), lambda i: (i, 0)),
        out_shape=jax.ShapeDtypeStruct((N, HID), jnp.float32),
    )(x, Wb, bb2)


def _hist_body(tids_hbm, counts_hbm, rank_hbm, tids_v, rank_v, cnt_vv):
    wid = lax.axis_index("c") * NS + lax.axis_index("s")
    base = wid * CHUNK
    lanes = lax.iota(jnp.int32, L)

    pltpu.sync_copy(tids_hbm.at[pl.ds(base, CHUNK)], tids_v)

    cnt = jnp.zeros((L,), jnp.int32)
    for g in range(CHUNK // L):
        tvec = tids_v[pl.ds(g * L, L)]
        rank_acc = jnp.zeros((L,), jnp.int32)
        for l in range(L):
            tid = tvec[l]
            m = lanes == tid
            r = jnp.sum(jnp.where(m, cnt, 0))
            rank_acc = jnp.where(lanes == l, r, rank_acc)
            cnt = cnt + jnp.where(m, 1, 0)
        rank_v[pl.ds(g * L, L)] = rank_acc
    cnt_vv[...] = cnt
    pltpu.sync_copy(cnt_vv, counts_hbm.at[wid])
    pltpu.sync_copy(rank_v, rank_hbm.at[pl.ds(base, CHUNK)])


def _hist(task_ids):
    f = functools.partial(
        pl.kernel,
        mesh=_mesh(),
        compiler_params=_SC_PARAMS,
        out_type=[jax.ShapeDtypeStruct((NW, L), jnp.int32),
                  jax.ShapeDtypeStruct((N,), jnp.int32)],
        scratch_types=[
            pltpu.VMEM((CHUNK,), jnp.int32),
            pltpu.VMEM((CHUNK,), jnp.int32),
            pltpu.VMEM((L,), jnp.int32),
        ],
    )(_hist_body)
    return f(task_ids)


def _global_offsets(cnt32_v, wid):
    tot = jnp.zeros((L,), jnp.int32)
    pre = jnp.zeros((L,), jnp.int32)
    for i in range(NW):
        row = cnt32_v[i]
        tot = tot + row
        pre = pre + row * (i < wid).astype(jnp.int32)
    cap = ((tot + (B - 1)) // B) * B
    start = plsc.cumsum(cap) - cap
    return start, cap, start + pre


def _scatter_body(feats_hbm, tids_hbm, counts_hbm, rank_hbm,
                  fs_hbm, pos_hbm, bt_hbm,
                  tids_v, rank_v, pos_v, base_v, cnt32_v, bt_v, bufs,
                  gsem0, gsem1, gsem2, gsem3, ssem):
    wid = lax.axis_index("c") * NS + lax.axis_index("s")
    base = wid * CHUNK
    lanes = lax.iota(jnp.int32, L)

    gsems = (gsem0, gsem1, gsem2, gsem3)
    pltpu.sync_copy(counts_hbm, cnt32_v)
    pltpu.sync_copy(tids_hbm.at[pl.ds(base, CHUNK)], tids_v)
    pltpu.sync_copy(rank_hbm.at[pl.ds(base, CHUNK)], rank_v)
    start, cap, wbase = _global_offsets(cnt32_v, wid)
    base_v[...] = wbase

    for g in range(CHUNK // L):
        idx = tids_v[pl.ds(g * L, L)]
        r = rank_v[pl.ds(g * L, L)]
        pos_v[pl.ds(g * L, L)] = plsc.load_gather(base_v, [idx]) + r
    pltpu.sync_copy(pos_v, pos_hbm.at[pl.ds(base, CHUNK)])

    gcp = [pltpu.async_copy(feats_hbm.at[pl.ds(base + g * L, L)],
                            bufs.at[g], gsems[g])
           for g in range(CHUNK // L)]
    scp = []
    for g in range(CHUNK // L):
        gcp[g].wait()
        idx = pos_v[pl.ds(g * L, L)]
        scp.append(pltpu.async_copy(bufs.at[g], fs_hbm.at[idx], ssem))
    for c in scp:
        c.wait()

    blk_start = start // B
    blk_end = blk_start + cap // B
    for h in range(BT_LEN // L):
        bvec = lanes + h * L
        btvec = jnp.zeros((L,), jnp.int32)
        for t in range(T):
            hit = (bvec >= blk_start[t]) & (bvec < blk_end[t])
            btvec = jnp.where(hit, t, btvec)
        bt_v[pl.ds(h * L, L)] = btvec

    @pl.when(wid == 0)
    def _():
        pltpu.sync_copy(bt_v, bt_hbm)


def _scatter(feats, tids, counts, rank):
    f = functools.partial(
        pl.kernel,
        mesh=_mesh(),
        compiler_params=_SC_PARAMS,
        out_type=[jax.ShapeDtypeStruct((P, HID), jnp.float32),
                  jax.ShapeDtypeStruct((N,), jnp.int32),
                  jax.ShapeDtypeStruct((BT_LEN,), jnp.int32)],
        scratch_types=[
            pltpu.VMEM((CHUNK,), jnp.int32),
            pltpu.VMEM((CHUNK,), jnp.int32),
            pltpu.VMEM((CHUNK,), jnp.int32),
            pltpu.VMEM((L,), jnp.int32),
            pltpu.VMEM((NW, L), jnp.int32),
            pltpu.VMEM((BT_LEN,), jnp.int32),
            pltpu.VMEM((CHUNK // L, L, HID), jnp.float32),
            pltpu.SemaphoreType.DMA,
            pltpu.SemaphoreType.DMA,
            pltpu.SemaphoreType.DMA,
            pltpu.SemaphoreType.DMA,
            pltpu.SemaphoreType.DMA,
        ],
    )(_scatter_body)
    return f(feats, tids, counts, rank)


def _heads_body(bt_ref, fs_ref, w1_ref, b1_ref, w2_ref, b2_ref, wo_ref, out_ref):
    xb = fs_ref[...].astype(jnp.bfloat16)
    h1 = jnp.dot(xb, w1_ref[0], preferred_element_type=jnp.float32)
    h1 = jnp.maximum(h1 + b1_ref[0], 0.0)
    h2 = jnp.dot(h1, w2_ref[0], preferred_element_type=jnp.float32)
    h2 = jnp.maximum(h2 + b2_ref[0], 0.0)
    o = lax.dot_general(wo_ref[0], h2, (((1,), (1,)), ((), ())),
                        preferred_element_type=jnp.float32)
    out_ref[0] = jnp.broadcast_to(o, (8, B))


def _heads(block_task, feats_s, W1, b1, W2p, b2a, woa):
    grid_spec = pltpu.PrefetchScalarGridSpec(
        num_scalar_prefetch=1,
        grid=(NBLK,),
        in_specs=[
            pl.BlockSpec((B, HID), lambda b, bt: (b, 0)),
            pl.BlockSpec((1, HID, 128), lambda b, bt: (bt[b], 0, 0)),
            pl.BlockSpec((1, 1, 128), lambda b, bt: (bt[b], 0, 0)),
            pl.BlockSpec((1, 128, 128), lambda b, bt: (bt[b], 0, 0)),
            pl.BlockSpec((1, 1, 128), lambda b, bt: (bt[b], 0, 0)),
            pl.BlockSpec((1, 1, 128), lambda b, bt: (bt[b], 0, 0)),
        ],
        out_specs=pl.BlockSpec((1, 8, B), lambda b, bt: (b, 0, 0)),
    )
    return pl.pallas_call(
        _heads_body,
        grid_spec=grid_spec,
        out_shape=jax.ShapeDtypeStruct((NBLK, 8, B), jnp.float32),
    )(block_task, feats_s, W1, b1, W2p, b2a, woa)


def _unsort_body(os_hbm, pos_hbm, out_hbm, bufs, pos_v, out_v,
                 sem0, sem1, sem2, sem3):
    wid = lax.axis_index("c") * NS + lax.axis_index("s")
    base = wid * CHUNK
    sems = (sem0, sem1, sem2, sem3)
    lanes = lax.iota(jnp.int32, L)
    pltpu.sync_copy(pos_hbm.at[pl.ds(base, CHUNK)], pos_v)
    cps = []
    for g in range(CHUNK // L):
        p = pos_v[pl.ds(g * L, L)]
        rowi = (p // B) * 8
        cps.append(pltpu.async_copy(os_hbm.at[rowi], bufs.at[g], sems[g]))
    for g in range(CHUNK // L):
        p = pos_v[pl.ds(g * L, L)]
        col = p % B
        cps[g].wait()
        out_v[pl.ds(g * L, L)] = plsc.load_gather(bufs.at[g], [lanes, col])
    pltpu.sync_copy(out_v, out_hbm.at[pl.ds(base, CHUNK)])


def _unsort(o_rows, pos):
    f = functools.partial(
        pl.kernel,
        mesh=_mesh(),
        compiler_params=_SC_PARAMS,
        out_type=jax.ShapeDtypeStruct((N,), jnp.float32),
        scratch_types=[
            pltpu.VMEM((CHUNK // L, L, B), jnp.float32),
            pltpu.VMEM((CHUNK,), jnp.int32),
            pltpu.VMEM((CHUNK,), jnp.float32),
            pltpu.SemaphoreType.DMA,
            pltpu.SemaphoreType.DMA,
            pltpu.SemaphoreType.DMA,
            pltpu.SemaphoreType.DMA,
        ],
    )(_unsort_body)
    return f(o_rows, pos)


def kernel(x, task_ids, Wb, bb, W1, b1, W2, b2, Wo, bo):
    tids = task_ids.astype(jnp.int32)
    bb2 = bb.reshape(1, HID)
    W2p = jnp.pad(W2, ((0, 0), (0, 0), (0, 128 - 64)))
    b1r = b1.reshape(T, 1, 128)
    b2a = jnp.concatenate(
        [b2, jnp.ones((T, 1), jnp.float32), jnp.zeros((T, 63), jnp.float32)],
        axis=1).reshape(T, 1, 128)
    woa = jnp.concatenate(
        [Wo[:, :, 0], bo, jnp.zeros((T, 63), jnp.float32)], axis=1).reshape(T, 1, 128)

    W1h = W1.astype(jnp.bfloat16)
    feats = _backbone(x, Wb, bb2)
    counts, rank = _hist(tids)
    feats_s, pos, block_task = _scatter(feats, tids, counts, rank)
    o_s = _heads(block_task, feats_s, W1h, b1r, W2p, b2a, woa)
    out = _unsort(o_s.reshape(NBLK * 8, B), pos)
    return out.reshape(N, 1)


# speedup vs baseline: 1.0271x; 1.0271x over previous
import functools

import jax
import jax.numpy as jnp
from jax import lax
from jax.experimental import pallas as pl
from jax.experimental.pallas import tpu as pltpu
from jax.experimental.pallas import tpu_sc as plsc

N = 2048
D_IN = 1024
HID = 1024
T = 16
B = 256
NBLK = 23
P = NBLK * B
BT_LEN = 32

NC, NS, L = 2, 16, 16
NW = NC * NS
CHUNK = N // NW


def _mesh():
    return plsc.VectorSubcoreMesh(core_axis_name="c", subcore_axis_name="s")


def _mesh1():
    return plsc.VectorSubcoreMesh(
        core_axis_name="c", subcore_axis_name="s", num_cores=1)


_SC_PARAMS = pltpu.CompilerParams(needs_layout_passes=False)


def _backbone_body(x_ref, wb_ref, bb_ref, out_ref):
    xb = x_ref[...].astype(jnp.bfloat16)
    wb = wb_ref[...].astype(jnp.bfloat16)
    acc = jnp.dot(xb, wb, preferred_element_type=jnp.float32)
    out_ref[...] = jnp.maximum(acc + bb_ref[...], 0.0)


def _backbone(x, Wb, bb2):
    return pl.pallas_call(
        _backbone_body,
        grid=(4,),
        in_specs=[
            pl.BlockSpec((N // 4, D_IN), lambda i: (i, 0)),
            pl.BlockSpec((D_IN, HID), lambda i: (0, 0)),
            pl.BlockSpec((1, HID), lambda i: (0, 0)),
        ],
        out_specs=pl.BlockSpec((N // 4, HID), lambda i: (i, 0)),
        out_shape=jax.ShapeDtypeStruct((N, HID), jnp.float32),
    )(x, Wb, bb2)


def _hist_body(tids_hbm, counts_hbm, rank_hbm, tids_v, rank_v, cnt_vv):
    wid = lax.axis_index("c") * NS + lax.axis_index("s")
    base = wid * CHUNK
    lanes = lax.iota(jnp.int32, L)

    pltpu.sync_copy(tids_hbm.at[pl.ds(base, CHUNK)], tids_v)

    cnt = jnp.zeros((L,), jnp.int32)
    for g in range(CHUNK // L):
        tvec = tids_v[pl.ds(g * L, L)]
        rank_acc = jnp.zeros((L,), jnp.int32)
        for l in range(L):
            tid = tvec[l]
            m = lanes == tid
            r = jnp.sum(jnp.where(m, cnt, 0))
            rank_acc = jnp.where(lanes == l, r, rank_acc)
            cnt = cnt + jnp.where(m, 1, 0)
        rank_v[pl.ds(g * L, L)] = rank_acc
    cnt_vv[...] = cnt
    pltpu.sync_copy(cnt_vv, counts_hbm.at[wid])
    pltpu.sync_copy(rank_v, rank_hbm.at[pl.ds(base, CHUNK)])


def _hist(task_ids):
    f = functools.partial(
        pl.kernel,
        mesh=_mesh(),
        compiler_params=_SC_PARAMS,
        out_type=[jax.ShapeDtypeStruct((NW, L), jnp.int32),
                  jax.ShapeDtypeStruct((N,), jnp.int32)],
        scratch_types=[
            pltpu.VMEM((CHUNK,), jnp.int32),
            pltpu.VMEM((CHUNK,), jnp.int32),
            pltpu.VMEM((L,), jnp.int32),
        ],
    )(_hist_body)
    return f(task_ids)


def _global_offsets(cnt32_v, wid):
    tot = jnp.zeros((L,), jnp.int32)
    pre = jnp.zeros((L,), jnp.int32)
    for i in range(NW):
        row = cnt32_v[i]
        tot = tot + row
        pre = pre + row * (i < wid).astype(jnp.int32)
    cap = ((tot + (B - 1)) // B) * B
    start = plsc.cumsum(cap) - cap
    return start, cap, start + pre


def _scatter_body(feats_hbm, tids_hbm, counts_hbm, rank_hbm,
                  fs_hbm, pos_hbm, bt_hbm,
                  tids_v, rank_v, pos_v, base_v, cnt32_v, bt_v, bufs, ssem):
    wid = lax.axis_index("c") * NS + lax.axis_index("s")
    base = wid * CHUNK
    lanes = lax.iota(jnp.int32, L)

    pltpu.sync_copy(counts_hbm, cnt32_v)
    pltpu.sync_copy(tids_hbm.at[pl.ds(base, CHUNK)], tids_v)
    pltpu.sync_copy(rank_hbm.at[pl.ds(base, CHUNK)], rank_v)
    start, cap, wbase = _global_offsets(cnt32_v, wid)
    base_v[...] = wbase

    for g in range(CHUNK // L):
        idx = tids_v[pl.ds(g * L, L)]
        r = rank_v[pl.ds(g * L, L)]
        pos_v[pl.ds(g * L, L)] = plsc.load_gather(base_v, [idx]) + r
    pltpu.sync_copy(pos_v, pos_hbm.at[pl.ds(base, CHUNK)])

    pltpu.sync_copy(feats_hbm.at[pl.ds(base, CHUNK)], bufs)
    pltpu.async_copy(bufs, fs_hbm.at[pos_v], ssem).wait()

    blk_start = start // B
    blk_end = blk_start + cap // B
    for h in range(BT_LEN // L):
        bvec = lanes + h * L
        btvec = jnp.zeros((L,), jnp.int32)
        for t in range(T):
            hit = (bvec >= blk_start[t]) & (bvec < blk_end[t])
            btvec = jnp.where(hit, t, btvec)
        bt_v[pl.ds(h * L, L)] = btvec

    @pl.when(wid == 0)
    def _():
        pltpu.sync_copy(bt_v, bt_hbm)


def _scatter(feats, tids, counts, rank):
    f = functools.partial(
        pl.kernel,
        mesh=_mesh(),
        compiler_params=_SC_PARAMS,
        out_type=[jax.ShapeDtypeStruct((P, HID), jnp.float32),
                  jax.ShapeDtypeStruct((N,), jnp.int32),
                  jax.ShapeDtypeStruct((BT_LEN,), jnp.int32)],
        scratch_types=[
            pltpu.VMEM((CHUNK,), jnp.int32),
            pltpu.VMEM((CHUNK,), jnp.int32),
            pltpu.VMEM((CHUNK,), jnp.int32),
            pltpu.VMEM((L,), jnp.int32),
            pltpu.VMEM((NW, L), jnp.int32),
            pltpu.VMEM((BT_LEN,), jnp.int32),
            pltpu.VMEM((CHUNK, HID), jnp.float32),
            pltpu.SemaphoreType.DMA,
        ],
    )(_scatter_body)
    return f(feats, tids, counts, rank)


def _heads_body(bt_ref, fs_ref, w1_ref, b1_ref, w2_ref, b2_ref, wo_ref, out_ref):
    xb = fs_ref[...].astype(jnp.bfloat16)
    h1 = jnp.dot(xb, w1_ref[0], preferred_element_type=jnp.float32)
    h1 = jnp.maximum(h1 + b1_ref[0], 0.0)
    h2 = jnp.dot(h1, w2_ref[0], preferred_element_type=jnp.float32)
    h2 = jnp.maximum(h2 + b2_ref[0], 0.0)
    o = lax.dot_general(wo_ref[0], h2, (((1,), (1,)), ((), ())),
                        preferred_element_type=jnp.float32)
    out_ref[0] = jnp.broadcast_to(o, (8, B))


def _heads(block_task, feats_s, W1, b1, W2p, b2a, woa):
    grid_spec = pltpu.PrefetchScalarGridSpec(
        num_scalar_prefetch=1,
        grid=(NBLK,),
        in_specs=[
            pl.BlockSpec((B, HID), lambda b, bt: (b, 0)),
            pl.BlockSpec((1, HID, 128), lambda b, bt: (bt[b], 0, 0)),
            pl.BlockSpec((1, 1, 128), lambda b, bt: (bt[b], 0, 0)),
            pl.BlockSpec((1, 128, 128), lambda b, bt: (bt[b], 0, 0)),
            pl.BlockSpec((1, 1, 128), lambda b, bt: (bt[b], 0, 0)),
            pl.BlockSpec((1, 1, 128), lambda b, bt: (bt[b], 0, 0)),
        ],
        out_specs=pl.BlockSpec((1, 8, B), lambda b, bt: (b, 0, 0)),
    )
    return pl.pallas_call(
        _heads_body,
        grid_spec=grid_spec,
        out_shape=jax.ShapeDtypeStruct((NBLK, 8, B), jnp.float32),
    )(block_task, feats_s, W1, b1, W2p, b2a, woa)


def _unsort_body(os_hbm, pos_hbm, out_hbm, bufs, pos_v, rows_v, out_v, sem):
    wid = lax.axis_index("c") * NS + lax.axis_index("s")
    base = wid * CHUNK
    lanes = lax.iota(jnp.int32, L)
    pltpu.sync_copy(pos_hbm.at[pl.ds(base, CHUNK)], pos_v)
    for g in range(CHUNK // L):
        p = pos_v[pl.ds(g * L, L)]
        rows_v[pl.ds(g * L, L)] = (p // B) * 8
    pltpu.async_copy(os_hbm.at[rows_v], bufs, sem).wait()
    for g in range(CHUNK // L):
        p = pos_v[pl.ds(g * L, L)]
        out_v[pl.ds(g * L, L)] = plsc.load_gather(
            bufs, [lanes + g * L, p % B])
    pltpu.sync_copy(out_v, out_hbm.at[pl.ds(base, CHUNK)])


def _unsort(o_rows, pos):
    f = functools.partial(
        pl.kernel,
        mesh=_mesh(),
        compiler_params=_SC_PARAMS,
        out_type=jax.ShapeDtypeStruct((N,), jnp.float32),
        scratch_types=[
            pltpu.VMEM((CHUNK, B), jnp.float32),
            pltpu.VMEM((CHUNK,), jnp.int32),
            pltpu.VMEM((CHUNK,), jnp.int32),
            pltpu.VMEM((CHUNK,), jnp.float32),
            pltpu.SemaphoreType.DMA,
        ],
    )(_unsort_body)
    return f(o_rows, pos)


def kernel(x, task_ids, Wb, bb, W1, b1, W2, b2, Wo, bo):
    tids = task_ids.astype(jnp.int32)
    bb2 = bb.reshape(1, HID)
    W2p = jnp.pad(W2, ((0, 0), (0, 0), (0, 128 - 64)))
    b1r = b1.reshape(T, 1, 128)
    b2a = jnp.concatenate(
        [b2, jnp.ones((T, 1), jnp.float32), jnp.zeros((T, 63), jnp.float32)],
        axis=1).reshape(T, 1, 128)
    woa = jnp.concatenate(
        [Wo[:, :, 0], bo, jnp.zeros((T, 63), jnp.float32)], axis=1).reshape(T, 1, 128)

    W1h = W1.astype(jnp.bfloat16)
    feats = _backbone(x, Wb, bb2)
    counts, rank = _hist(tids)
    feats_s, pos, block_task = _scatter(feats, tids, counts, rank)
    o_s = _heads(block_task, feats_s, W1h, b1r, W2p, b2a, woa)
    out = _unsort(o_s.reshape(NBLK * 8, B), pos)
    return out.reshape(N, 1)


# speedup vs baseline: 1.0451x; 1.0175x over previous
import functools

import jax
import jax.numpy as jnp
from jax import lax
from jax.experimental import pallas as pl
from jax.experimental.pallas import tpu as pltpu
from jax.experimental.pallas import tpu_sc as plsc

N = 2048
D_IN = 1024
HID = 1024
T = 16
B = 256
NBLK = 23
P = NBLK * B
BT_LEN = 32

NC, NS, L = 2, 16, 16
NW = NC * NS
CHUNK = N // NW


def _mesh():
    return plsc.VectorSubcoreMesh(core_axis_name="c", subcore_axis_name="s")


def _mesh1():
    return plsc.VectorSubcoreMesh(
        core_axis_name="c", subcore_axis_name="s", num_cores=1)


_SC_PARAMS = pltpu.CompilerParams(needs_layout_passes=False)


def _backbone_body(x_ref, wb_ref, bb_ref, out_ref):
    acc = jnp.dot(x_ref[...], wb_ref[...], preferred_element_type=jnp.float32)
    out_ref[...] = jnp.maximum(acc + bb_ref[...], 0.0)


def _backbone(x, Wb, bb2):
    return pl.pallas_call(
        _backbone_body,
        grid=(4,),
        in_specs=[
            pl.BlockSpec((N // 4, D_IN), lambda i: (i, 0)),
            pl.BlockSpec((D_IN, HID), lambda i: (0, 0)),
            pl.BlockSpec((1, HID), lambda i: (0, 0)),
        ],
        out_specs=pl.BlockSpec((N // 4, HID), lambda i: (i, 0)),
        out_shape=jax.ShapeDtypeStruct((N, HID), jnp.float32),
    )(x, Wb, bb2)


def _hist_body(tids_hbm, counts_hbm, rank_hbm, tids_v, rank_v, cnt_vv):
    wid = lax.axis_index("c") * NS + lax.axis_index("s")
    base = wid * CHUNK
    lanes = lax.iota(jnp.int32, L)

    pltpu.sync_copy(tids_hbm.at[pl.ds(base, CHUNK)], tids_v)

    cnt = jnp.zeros((L,), jnp.int32)
    for g in range(CHUNK // L):
        tvec = tids_v[pl.ds(g * L, L)]
        rank_acc = jnp.zeros((L,), jnp.int32)
        for l in range(L):
            tid = tvec[l]
            m = lanes == tid
            r = jnp.sum(jnp.where(m, cnt, 0))
            rank_acc = jnp.where(lanes == l, r, rank_acc)
            cnt = cnt + jnp.where(m, 1, 0)
        rank_v[pl.ds(g * L, L)] = rank_acc
    cnt_vv[...] = cnt
    pltpu.sync_copy(cnt_vv, counts_hbm.at[wid])
    pltpu.sync_copy(rank_v, rank_hbm.at[pl.ds(base, CHUNK)])


def _hist(task_ids):
    f = functools.partial(
        pl.kernel,
        mesh=_mesh(),
        compiler_params=_SC_PARAMS,
        out_type=[jax.ShapeDtypeStruct((NW, L), jnp.int32),
                  jax.ShapeDtypeStruct((N,), jnp.int32)],
        scratch_types=[
            pltpu.VMEM((CHUNK,), jnp.int32),
            pltpu.VMEM((CHUNK,), jnp.int32),
            pltpu.VMEM((L,), jnp.int32),
        ],
    )(_hist_body)
    return f(task_ids)


def _global_offsets(cnt32_v, wid):
    tot = jnp.zeros((L,), jnp.int32)
    pre = jnp.zeros((L,), jnp.int32)
    for i in range(NW):
        row = cnt32_v[i]
        tot = tot + row
        pre = pre + row * (i < wid).astype(jnp.int32)
    cap = ((tot + (B - 1)) // B) * B
    start = plsc.cumsum(cap) - cap
    return start, cap, start + pre


def _scatter_body(feats_hbm, tids_hbm, counts_hbm, rank_hbm,
                  fs_hbm, pos_hbm, bt_hbm,
                  tids_v, rank_v, pos_v, base_v, cnt32_v, bt_v, bufs, ssem):
    wid = lax.axis_index("c") * NS + lax.axis_index("s")
    base = wid * CHUNK
    lanes = lax.iota(jnp.int32, L)

    pltpu.sync_copy(counts_hbm, cnt32_v)
    pltpu.sync_copy(tids_hbm.at[pl.ds(base, CHUNK)], tids_v)
    pltpu.sync_copy(rank_hbm.at[pl.ds(base, CHUNK)], rank_v)
    start, cap, wbase = _global_offsets(cnt32_v, wid)
    base_v[...] = wbase

    for g in range(CHUNK // L):
        idx = tids_v[pl.ds(g * L, L)]
        r = rank_v[pl.ds(g * L, L)]
        pos_v[pl.ds(g * L, L)] = plsc.load_gather(base_v, [idx]) + r
    pltpu.sync_copy(pos_v, pos_hbm.at[pl.ds(base, CHUNK)])

    pltpu.sync_copy(feats_hbm.at[pl.ds(base, CHUNK)], bufs)
    pltpu.async_copy(bufs, fs_hbm.at[pos_v], ssem).wait()

    blk_start = start // B
    blk_end = blk_start + cap // B
    for h in range(BT_LEN // L):
        bvec = lanes + h * L
        btvec = jnp.full((L,), -1, jnp.int32)
        for t in range(T):
            hit = (bvec >= blk_start[t]) & (bvec < blk_end[t])
            btvec = jnp.where(hit, t, btvec)
        bt_v[pl.ds(h * L, L)] = btvec

    @pl.when(wid == 0)
    def _():
        pltpu.sync_copy(bt_v, bt_hbm)


def _scatter(feats, tids, counts, rank):
    f = functools.partial(
        pl.kernel,
        mesh=_mesh(),
        compiler_params=_SC_PARAMS,
        out_type=[jax.ShapeDtypeStruct((P, HID), jnp.float32),
                  jax.ShapeDtypeStruct((N,), jnp.int32),
                  jax.ShapeDtypeStruct((BT_LEN,), jnp.int32)],
        scratch_types=[
            pltpu.VMEM((CHUNK,), jnp.int32),
            pltpu.VMEM((CHUNK,), jnp.int32),
            pltpu.VMEM((CHUNK,), jnp.int32),
            pltpu.VMEM((L,), jnp.int32),
            pltpu.VMEM((NW, L), jnp.int32),
            pltpu.VMEM((BT_LEN,), jnp.int32),
            pltpu.VMEM((CHUNK, HID), jnp.float32),
            pltpu.SemaphoreType.DMA,
        ],
    )(_scatter_body)
    return f(feats, tids, counts, rank)


def _heads_body(bt_ref, fs_ref, w1_ref, b1_ref, w2_ref, b2_ref, wo_ref, out_ref):
    xb = fs_ref[...].astype(jnp.bfloat16)
    h1 = jnp.dot(xb, w1_ref[0], preferred_element_type=jnp.float32)
    h1 = jnp.maximum(h1 + b1_ref[0], 0.0)
    h2 = jnp.dot(h1, w2_ref[0], preferred_element_type=jnp.float32)
    h2 = jnp.maximum(h2 + b2_ref[0], 0.0)
    o = lax.dot_general(wo_ref[0], h2, (((1,), (1,)), ((), ())),
                        preferred_element_type=jnp.float32)
    out_ref[...] = jnp.broadcast_to(o, (8, B))


def _heads(block_task, feats_s, W1, b1, W2p, b2a, woa):
    grid_spec = pltpu.PrefetchScalarGridSpec(
        num_scalar_prefetch=1,
        grid=(NBLK,),
        in_specs=[
            pl.BlockSpec((B, HID), lambda b, bt: (jnp.where(bt[b] < 0, 0, b), 0)),
            pl.BlockSpec((1, HID, 128), lambda b, bt: (jnp.maximum(bt[b], 0), 0, 0)),
            pl.BlockSpec((1, 1, 128), lambda b, bt: (jnp.maximum(bt[b], 0), 0, 0)),
            pl.BlockSpec((1, 128, 128), lambda b, bt: (jnp.maximum(bt[b], 0), 0, 0)),
            pl.BlockSpec((1, 1, 128), lambda b, bt: (jnp.maximum(bt[b], 0), 0, 0)),
            pl.BlockSpec((1, 1, 128), lambda b, bt: (jnp.maximum(bt[b], 0), 0, 0)),
        ],
        out_specs=pl.BlockSpec((8, B), lambda b, bt: (0, b)),
    )
    return pl.pallas_call(
        _heads_body,
        grid_spec=grid_spec,
        out_shape=jax.ShapeDtypeStruct((8, NBLK * B), jnp.float32),
    )(block_task, feats_s, W1, b1, W2p, b2a, woa)


def _unsort_body(os_hbm, pos_hbm, out_hbm, os_v, pos_v, out_v):
    wid = lax.axis_index("c") * NS + lax.axis_index("s")
    base = wid * CHUNK
    pltpu.sync_copy(os_hbm.at[0], os_v)
    pltpu.sync_copy(pos_hbm.at[pl.ds(base, CHUNK)], pos_v)
    for g in range(CHUNK // L):
        p = pos_v[pl.ds(g * L, L)]
        out_v[pl.ds(g * L, L)] = plsc.load_gather(os_v, [p])
    pltpu.sync_copy(out_v, out_hbm.at[pl.ds(base, CHUNK)])


def _unsort(o_rows, pos):
    f = functools.partial(
        pl.kernel,
        mesh=_mesh(),
        compiler_params=_SC_PARAMS,
        out_type=jax.ShapeDtypeStruct((N,), jnp.float32),
        scratch_types=[
            pltpu.VMEM((P,), jnp.float32),
            pltpu.VMEM((CHUNK,), jnp.int32),
            pltpu.VMEM((CHUNK,), jnp.float32),
        ],
    )(_unsort_body)
    return f(o_rows, pos)


def kernel(x, task_ids, Wb, bb, W1, b1, W2, b2, Wo, bo):
    tids = task_ids.astype(jnp.int32)
    bb2 = bb.reshape(1, HID)
    W2p = jnp.pad(W2, ((0, 0), (0, 0), (0, 128 - 64)))
    b1r = b1.reshape(T, 1, 128)
    b2a = jnp.concatenate(
        [b2, jnp.ones((T, 1), jnp.float32), jnp.zeros((T, 63), jnp.float32)],
        axis=1).reshape(T, 1, 128)
    woa = jnp.concatenate(
        [Wo[:, :, 0], bo, jnp.zeros((T, 63), jnp.float32)], axis=1).reshape(T, 1, 128)

    W1h = W1.astype(jnp.bfloat16)
    feats = _backbone(x.astype(jnp.bfloat16), Wb.astype(jnp.bfloat16), bb2)
    counts, rank = _hist(tids)
    feats_s, pos, block_task = _scatter(feats, tids, counts, rank)
    o_s = _heads(block_task, feats_s, W1h, b1r, W2p, b2a, woa)
    out = _unsort(o_s, pos)
    return out.reshape(N, 1)


# speedup vs baseline: 1.1818x; 1.1309x over previous
import functools

import jax
import jax.numpy as jnp
from jax import lax
from jax.experimental import pallas as pl
from jax.experimental.pallas import tpu as pltpu
from jax.experimental.pallas import tpu_sc as plsc

N = 2048
D_IN = 1024
HID = 1024
T = 16
B = 256
NBLK = 23
P = NBLK * B
BT_LEN = 32

NC, NS, L = 2, 16, 16
NW = NC * NS
CHUNK = N // NW


def _mesh():
    return plsc.VectorSubcoreMesh(core_axis_name="c", subcore_axis_name="s")


def _mesh1():
    return plsc.VectorSubcoreMesh(
        core_axis_name="c", subcore_axis_name="s", num_cores=1)


_SC_PARAMS = pltpu.CompilerParams(needs_layout_passes=False)


def _backbone_body(x_ref, wb_ref, bb_ref, out_ref):
    xb = x_ref[...].astype(jnp.bfloat16)
    wb = wb_ref[...].astype(jnp.bfloat16)
    acc = jnp.dot(xb, wb, preferred_element_type=jnp.float32)
    out_ref[...] = jnp.maximum(acc + bb_ref[...], 0.0)


def _backbone(x, Wb, bb2):
    return pl.pallas_call(
        _backbone_body,
        grid=(4,),
        in_specs=[
            pl.BlockSpec((N // 4, D_IN), lambda i: (i, 0)),
            pl.BlockSpec((D_IN, HID), lambda i: (0, 0)),
            pl.BlockSpec((1, HID), lambda i: (0, 0)),
        ],
        out_specs=pl.BlockSpec((N // 4, HID), lambda i: (i, 0)),
        out_shape=jax.ShapeDtypeStruct((N, HID), jnp.float32),
    )(x, Wb, bb2)


def _hist_body(tids_hbm, counts_hbm, rank_hbm, tids_v, rank_v, cnt_vv):
    wid = lax.axis_index("c") * NS + lax.axis_index("s")
    base = wid * CHUNK
    lanes = lax.iota(jnp.int32, L)

    pltpu.sync_copy(tids_hbm.at[pl.ds(base, CHUNK)], tids_v)

    cnt = jnp.zeros((L,), jnp.int32)
    for g in range(CHUNK // L):
        tvec = tids_v[pl.ds(g * L, L)]
        rank_acc = jnp.zeros((L,), jnp.int32)
        for l in range(L):
            tid = tvec[l]
            m = lanes == tid
            r = jnp.sum(jnp.where(m, cnt, 0))
            rank_acc = jnp.where(lanes == l, r, rank_acc)
            cnt = cnt + jnp.where(m, 1, 0)
        rank_v[pl.ds(g * L, L)] = rank_acc
    cnt_vv[...] = cnt
    pltpu.sync_copy(cnt_vv, counts_hbm.at[wid])
    pltpu.sync_copy(rank_v, rank_hbm.at[pl.ds(base, CHUNK)])


def _hist(task_ids):
    f = functools.partial(
        pl.kernel,
        mesh=_mesh(),
        compiler_params=_SC_PARAMS,
        out_type=[jax.ShapeDtypeStruct((NW, L), jnp.int32),
                  jax.ShapeDtypeStruct((N,), jnp.int32)],
        scratch_types=[
            pltpu.VMEM((CHUNK,), jnp.int32),
            pltpu.VMEM((CHUNK,), jnp.int32),
            pltpu.VMEM((L,), jnp.int32),
        ],
    )(_hist_body)
    return f(task_ids)


def _global_offsets(cnt32_v, wid):
    tot = jnp.zeros((L,), jnp.int32)
    pre = jnp.zeros((L,), jnp.int32)
    for i in range(NW):
        row = cnt32_v[i]
        tot = tot + row
        pre = pre + row * (i < wid).astype(jnp.int32)
    cap = ((tot + (B - 1)) // B) * B
    start = plsc.cumsum(cap) - cap
    return start, cap, start + pre


def _scatter_body(feats_hbm, tids_hbm, counts_hbm, rank_hbm,
                  fs_hbm, pos_hbm, bt_hbm,
                  tids_v, rank_v, pos_v, base_v, cnt32_v, bt_v, bufs, ssem):
    wid = lax.axis_index("c") * NS + lax.axis_index("s")
    base = wid * CHUNK
    lanes = lax.iota(jnp.int32, L)

    pltpu.sync_copy(counts_hbm, cnt32_v)
    pltpu.sync_copy(tids_hbm.at[pl.ds(base, CHUNK)], tids_v)
    pltpu.sync_copy(rank_hbm.at[pl.ds(base, CHUNK)], rank_v)
    start, cap, wbase = _global_offsets(cnt32_v, wid)
    base_v[...] = wbase

    for g in range(CHUNK // L):
        idx = tids_v[pl.ds(g * L, L)]
        r = rank_v[pl.ds(g * L, L)]
        pos_v[pl.ds(g * L, L)] = plsc.load_gather(base_v, [idx]) + r
    pltpu.sync_copy(pos_v, pos_hbm.at[pl.ds(base, CHUNK)])

    pltpu.sync_copy(feats_hbm.at[pl.ds(base, CHUNK)], bufs)
    pltpu.async_copy(bufs, fs_hbm.at[pos_v], ssem).wait()

    blk_start = start // B
    blk_end = blk_start + cap // B
    for h in range(BT_LEN // L):
        bvec = lanes + h * L
        btvec = jnp.full((L,), -1, jnp.int32)
        for t in range(T):
            hit = (bvec >= blk_start[t]) & (bvec < blk_end[t])
            btvec = jnp.where(hit, t, btvec)
        bt_v[pl.ds(h * L, L)] = btvec

    @pl.when(wid == 0)
    def _():
        pltpu.sync_copy(bt_v, bt_hbm)


def _scatter(feats, tids, counts, rank):
    f = functools.partial(
        pl.kernel,
        mesh=_mesh(),
        compiler_params=_SC_PARAMS,
        out_type=[jax.ShapeDtypeStruct((P, HID), jnp.float32),
                  jax.ShapeDtypeStruct((N,), jnp.int32),
                  jax.ShapeDtypeStruct((BT_LEN,), jnp.int32)],
        scratch_types=[
            pltpu.VMEM((CHUNK,), jnp.int32),
            pltpu.VMEM((CHUNK,), jnp.int32),
            pltpu.VMEM((CHUNK,), jnp.int32),
            pltpu.VMEM((L,), jnp.int32),
            pltpu.VMEM((NW, L), jnp.int32),
            pltpu.VMEM((BT_LEN,), jnp.int32),
            pltpu.VMEM((CHUNK, HID), jnp.float32),
            pltpu.SemaphoreType.DMA,
        ],
    )(_scatter_body)
    return f(feats, tids, counts, rank)


def _heads_body(bt_ref, fs_ref, w1_ref, b1_ref, w2_ref, b2_ref, wo_ref, out_ref):
    xb = fs_ref[...].astype(jnp.bfloat16)
    h1 = jnp.dot(xb, w1_ref[0].astype(jnp.bfloat16),
                 preferred_element_type=jnp.float32)
    h1 = jnp.maximum(h1 + b1_ref[0], 0.0)
    h2 = jnp.dot(h1, w2_ref[0], preferred_element_type=jnp.float32)
    h2 = jnp.maximum(h2 + b2_ref[0], 0.0)
    o = lax.dot_general(wo_ref[0], h2, (((1,), (1,)), ((), ())),
                        preferred_element_type=jnp.float32)
    out_ref[...] = jnp.broadcast_to(o, (8, B))


def _heads(block_task, feats_s, W1, b1, W2p, b2a, woa):
    grid_spec = pltpu.PrefetchScalarGridSpec(
        num_scalar_prefetch=1,
        grid=(NBLK,),
        in_specs=[
            pl.BlockSpec((B, HID), lambda b, bt: (jnp.where(bt[b] < 0, 0, b), 0)),
            pl.BlockSpec((1, HID, 128), lambda b, bt: (jnp.maximum(bt[b], 0), 0, 0)),
            pl.BlockSpec((1, 1, 128), lambda b, bt: (jnp.maximum(bt[b], 0), 0, 0)),
            pl.BlockSpec((1, 128, 128), lambda b, bt: (jnp.maximum(bt[b], 0), 0, 0)),
            pl.BlockSpec((1, 1, 128), lambda b, bt: (jnp.maximum(bt[b], 0), 0, 0)),
            pl.BlockSpec((1, 1, 128), lambda b, bt: (jnp.maximum(bt[b], 0), 0, 0)),
        ],
        out_specs=pl.BlockSpec((8, B), lambda b, bt: (0, b)),
    )
    return pl.pallas_call(
        _heads_body,
        grid_spec=grid_spec,
        out_shape=jax.ShapeDtypeStruct((8, NBLK * B), jnp.float32),
    )(block_task, feats_s, W1, b1, W2p, b2a, woa)


def _unsort_body(os_hbm, pos_hbm, out_hbm, os_v, pos_v, out_v):
    wid = lax.axis_index("c") * NS + lax.axis_index("s")
    base = wid * CHUNK
    pltpu.sync_copy(os_hbm.at[0], os_v)
    pltpu.sync_copy(pos_hbm.at[pl.ds(base, CHUNK)], pos_v)
    for g in range(CHUNK // L):
        p = pos_v[pl.ds(g * L, L)]
        out_v[pl.ds(g * L, L)] = plsc.load_gather(os_v, [p])
    pltpu.sync_copy(out_v, out_hbm.at[pl.ds(base, CHUNK)])


def _unsort(o_rows, pos):
    f = functools.partial(
        pl.kernel,
        mesh=_mesh(),
        compiler_params=_SC_PARAMS,
        out_type=jax.ShapeDtypeStruct((N,), jnp.float32),
        scratch_types=[
            pltpu.VMEM((P,), jnp.float32),
            pltpu.VMEM((CHUNK,), jnp.int32),
            pltpu.VMEM((CHUNK,), jnp.float32),
        ],
    )(_unsort_body)
    return f(o_rows, pos)


def kernel(x, task_ids, Wb, bb, W1, b1, W2, b2, Wo, bo):
    tids = task_ids.astype(jnp.int32)
    bb2 = bb.reshape(1, HID)
    W2p = jnp.pad(W2, ((0, 0), (0, 0), (0, 128 - 64)))
    b1r = b1.reshape(T, 1, 128)
    b2a = jnp.concatenate(
        [b2, jnp.ones((T, 1), jnp.float32), jnp.zeros((T, 63), jnp.float32)],
        axis=1).reshape(T, 1, 128)
    woa = jnp.concatenate(
        [Wo[:, :, 0], bo, jnp.zeros((T, 63), jnp.float32)], axis=1).reshape(T, 1, 128)

    feats = _backbone(x, Wb, bb2)
    counts, rank = _hist(tids)
    feats_s, pos, block_task = _scatter(feats, tids, counts, rank)
    o_s = _heads(block_task, feats_s, W1, b1r, W2p, b2a, woa)
    out = _unsort(o_s, pos)
    return out.reshape(N, 1)
